# speedup vs baseline: 1.2776x; 1.2776x over previous
import functools

import jax
import jax.numpy as jnp
from jax import lax
from jax.experimental import pallas as pl
from jax.experimental.pallas import tpu as pltpu
from jax.experimental.pallas import tpu_sc as plsc


def _embed_lookup(ids_flat, tok_embed, pos_embed, seq_len):
    B = ids_flat.shape[0]
    _, d = tok_embed.shape
    info = plsc.get_sparse_core_info()
    num_workers = info.num_cores * info.num_subcores
    b_per_w = B // num_workers
    mesh = plsc.VectorSubcoreMesh(core_axis_name="c", subcore_axis_name="s")

    @functools.partial(
        pl.kernel,
        mesh=mesh,
        out_type=jax.ShapeDtypeStruct((B, d), jnp.float32),
        scratch_types=[
            pltpu.VMEM((b_per_w,), jnp.int32),
            pltpu.VMEM((b_per_w, d), jnp.float32),
            pltpu.VMEM((b_per_w, d), jnp.float32),
            pltpu.SemaphoreType.DMA,
        ],
    )
    def _emb(ids_hbm, tok_hbm, pos_hbm, out_hbm, idx_v, tok_v, pos_v, sem):
        wid = lax.axis_index("s") * info.num_cores + lax.axis_index("c")
        base = pl.multiple_of(wid * b_per_w, b_per_w)
        pos_base = pl.multiple_of(lax.rem(base, seq_len), b_per_w)

        pltpu.sync_copy(ids_hbm.at[pl.ds(base, b_per_w)], idx_v)
        gather = pltpu.async_copy(tok_hbm.at[idx_v], tok_v, sem)
        pltpu.sync_copy(pos_hbm.at[pl.ds(pos_base, b_per_w)], pos_v)
        gather.wait()

        def row(i, carry):
            for j in range(d // 16):
                sl = pl.ds(j * 16, 16)
                tok_v[i, sl] = tok_v[i, sl] + pos_v[i, sl]
            return carry

        lax.fori_loop(0, b_per_w, row, 0)
        pltpu.sync_copy(tok_v, out_hbm.at[pl.ds(base, b_per_w)])

    return _emb(ids_flat, tok_embed, pos_embed)


def kernel(ids, tok_embed, pos_embed):
    batch, seq_len = ids.shape
    _, d = tok_embed.shape
    ids_flat = ids.reshape(batch * seq_len).astype(jnp.int32)
    out = _embed_lookup(ids_flat, tok_embed, pos_embed, seq_len)
    return out.reshape(batch, seq_len, d)


# speedup vs baseline: 1.3354x; 1.0452x over previous
import functools

import jax
import jax.numpy as jnp
from jax import lax
from jax.experimental import pallas as pl
from jax.experimental.pallas import tpu as pltpu
from jax.experimental.pallas import tpu_sc as plsc


def _embed_lookup(ids_flat, tok_embed, pos_embed, batch, seq_len):
    B = ids_flat.shape[0]
    _, d = tok_embed.shape
    info = plsc.get_sparse_core_info()
    num_workers = info.num_cores * info.num_subcores
    s_per_w = seq_len // num_workers
    mesh = plsc.VectorSubcoreMesh(core_axis_name="c", subcore_axis_name="s")

    @functools.partial(
        pl.kernel,
        mesh=mesh,
        out_type=jax.ShapeDtypeStruct((B, d), jnp.float32),
        scratch_types=[
            pltpu.VMEM((batch * s_per_w,), jnp.int32),
            pltpu.VMEM((batch, s_per_w, d), jnp.float32),
            pltpu.VMEM((s_per_w, d), jnp.float32),
            pltpu.SemaphoreType.DMA,
            pltpu.SemaphoreType.DMA((batch,)),
            pltpu.SemaphoreType.DMA,
        ],
    )
    def _emb(ids_hbm, tok_hbm, pos_hbm, out_hbm, idx_v, tok_v, pos_v,
             sem_in, sem_g, sem_st):
        wid = lax.axis_index("s") * info.num_cores + lax.axis_index("c")
        sbase = pl.multiple_of(wid * s_per_w, s_per_w)

        idx_copies = []
        for b in range(batch):
            idx_copies.append(pltpu.async_copy(
                ids_hbm.at[pl.ds(b * seq_len + sbase, s_per_w)],
                idx_v.at[pl.ds(b * s_per_w, s_per_w)], sem_in))
        pos_copy = pltpu.async_copy(pos_hbm.at[pl.ds(sbase, s_per_w)],
                                    pos_v, sem_in)
        for c in idx_copies:
            c.wait()

        gathers = []
        for b in range(batch):
            gathers.append(pltpu.async_copy(
                tok_hbm.at[idx_v.at[pl.ds(b * s_per_w, s_per_w)]],
                tok_v.at[b], sem_g.at[b]))
        pos_copy.wait()

        stores = []
        for b in range(batch):
            gathers[b].wait()

            @plsc.parallel_loop(0, s_per_w, unroll=2)
            def _row(i, _b=b):
                for j in range(d // 16):
                    sl = pl.ds(j * 16, 16)
                    tok_v[_b, i, sl] = tok_v[_b, i, sl] + pos_v[i, sl]

            stores.append(pltpu.async_copy(
                tok_v.at[b],
                out_hbm.at[pl.ds(b * seq_len + sbase, s_per_w)], sem_st))
        for s in stores:
            s.wait()

    return _emb(ids_flat, tok_embed, pos_embed)


def kernel(ids, tok_embed, pos_embed):
    batch, seq_len = ids.shape
    _, d = tok_embed.shape
    ids_flat = ids.reshape(batch * seq_len).astype(jnp.int32)
    out = _embed_lookup(ids_flat, tok_embed, pos_embed, batch, seq_len)
    return out.reshape(batch, seq_len, d)


# speedup vs baseline: 1.3377x; 1.0017x over previous
import functools

import jax
import jax.numpy as jnp
from jax import lax
from jax.experimental import pallas as pl
from jax.experimental.pallas import tpu as pltpu
from jax.experimental.pallas import tpu_sc as plsc


def _embed_lookup(ids, tok_embed, pos_embed):
    batch, seq_len = ids.shape
    B = batch * seq_len
    _, d = tok_embed.shape
    info = plsc.get_sparse_core_info()
    num_workers = info.num_cores * info.num_subcores
    s_per_w = seq_len // num_workers
    mesh = plsc.VectorSubcoreMesh(core_axis_name="c", subcore_axis_name="s")

    @functools.partial(
        pl.kernel,
        mesh=mesh,
        out_type=jax.ShapeDtypeStruct((B, d), jnp.float32),
        scratch_types=[
            pltpu.VMEM((batch * s_per_w,), jnp.int32),
            pltpu.VMEM((batch, s_per_w, d), jnp.float32),
            pltpu.VMEM((s_per_w, d), jnp.float32),
            pltpu.SemaphoreType.DMA,
            pltpu.SemaphoreType.DMA((batch,)),
            pltpu.SemaphoreType.DMA,
        ],
    )
    def _emb(ids_hbm, tok_hbm, pos_hbm, out_hbm, idx_v, tok_v, pos_v,
             sem_in, sem_g, sem_st):
        wid = lax.axis_index("s") * info.num_cores + lax.axis_index("c")
        sbase = pl.multiple_of(wid * s_per_w, s_per_w)

        idx_copies = []
        for b in range(batch):
            idx_copies.append(pltpu.async_copy(
                ids_hbm.at[b, pl.ds(sbase, s_per_w)],
                idx_v.at[pl.ds(b * s_per_w, s_per_w)], sem_in))
        pos_copy = pltpu.async_copy(pos_hbm.at[pl.ds(sbase, s_per_w)],
                                    pos_v, sem_in)
        for c in idx_copies:
            c.wait()

        gathers = []
        for b in range(batch):
            gathers.append(pltpu.async_copy(
                tok_hbm.at[idx_v.at[pl.ds(b * s_per_w, s_per_w)]],
                tok_v.at[b], sem_g.at[b]))
        pos_copy.wait()

        stores = []
        for b in range(batch):
            gathers[b].wait()

            @plsc.parallel_loop(0, s_per_w, unroll=2)
            def _row(i, _b=b):
                for j in range(d // 16):
                    sl = pl.ds(j * 16, 16)
                    tok_v[_b, i, sl] = tok_v[_b, i, sl] + pos_v[i, sl]

            stores.append(pltpu.async_copy(
                tok_v.at[b],
                out_hbm.at[pl.ds(b * seq_len + sbase, s_per_w)], sem_st))
        for s in stores:
            s.wait()

    return _emb(ids, tok_embed, pos_embed)


def kernel(ids, tok_embed, pos_embed):
    batch, seq_len = ids.shape
    _, d = tok_embed.shape
    out = _embed_lookup(ids.astype(jnp.int32), tok_embed, pos_embed)
    return out.reshape(batch, seq_len, d)
